# Initial kernel scaffold; baseline (speedup 1.0000x reference)
import jax
import jax.numpy as jnp
from jax.experimental import pallas as pl


def kernel(scores, bbox_deltas, im_info):
    raise NotImplementedError("write your pallas kernel here")


# speedup vs baseline: 22.6587x; 22.6587x over previous
import numpy as np
import jax
import jax.numpy as jnp
from jax.experimental import pallas as pl
from jax.experimental.pallas import tpu as pltpu

_B = 4
_H = 64
_W = 64
_A = 9
_N = _H * _W * _A
_ROWS = 288
_LANES = 128
_PRE = 6000
_POST = 300
_THRESH = 0.7
_STRIDE = 16.0


def _gen_anchors():
    def whctrs(a):
        w = a[2] - a[0] + 1
        h = a[3] - a[1] + 1
        return w, h, a[0] + 0.5 * (w - 1), a[1] + 0.5 * (h - 1)

    def mk(ws, hs, xc, yc):
        ws = ws[:, None]
        hs = hs[:, None]
        return np.hstack((xc - 0.5 * (ws - 1), yc - 0.5 * (hs - 1),
                          xc + 0.5 * (ws - 1), yc + 0.5 * (hs - 1)))

    base = np.array([1, 1, 16, 16], dtype=np.float64) - 1
    ratios = np.array([0.5, 1.0, 2.0])
    scales = np.array([8, 16, 32])
    w, h, xc, yc = whctrs(base)
    size = w * h
    ws = np.round(np.sqrt(size / ratios))
    hs = np.round(ws * ratios)
    ra = mk(ws, hs, xc, yc)
    out = []
    for i in range(ra.shape[0]):
        w, h, xc, yc = whctrs(ra[i, :])
        out.append(mk(w * scales, h * scales, xc, yc))
    return np.vstack(out).astype(np.float32)


def _anchor_tiles():
    anch = _gen_anchors()
    sx = (np.arange(_W, dtype=np.float32) * np.float32(_STRIDE))
    sy = (np.arange(_H, dtype=np.float32) * np.float32(_STRIDE))
    syg, sxg = np.meshgrid(sy, sx, indexing="ij")
    shifts = np.stack([sxg.ravel(), syg.ravel(),
                       sxg.ravel(), syg.ravel()], axis=1).astype(np.float32)
    boxes = (anch[None, :, :] + shifts[:, None, :]).reshape(_N, 4)
    boxes = boxes.astype(np.float32)
    wa = boxes[:, 2] - boxes[:, 0] + np.float32(1.0)
    ha = boxes[:, 3] - boxes[:, 1] + np.float32(1.0)
    cxa = boxes[:, 0] + np.float32(0.5) * wa
    cya = boxes[:, 1] + np.float32(0.5) * ha
    r = lambda v: v.reshape(_ROWS, _LANES)
    return r(wa), r(ha), r(cxa), r(cya)


_WA, _HA, _CXA, _CYA = [jnp.asarray(t) for t in _anchor_tiles()]
_LT128 = jnp.asarray(np.triu(np.ones((_LANES, _LANES), np.float32), k=1))
_ONES128 = jnp.asarray(np.ones((_LANES, _LANES), np.float32))
_L288 = jnp.asarray(np.tril(np.ones((_ROWS, _ROWS), np.float32), k=-1))
_LIN = jnp.asarray(
    np.arange(_N, dtype=np.float32).reshape(_ROWS, _LANES))


def _nms_kernel(sb_ref, dx_ref, dy_ref, dw_ref, dh_ref, im_ref,
                wa_ref, ha_ref, cxa_ref, cya_ref,
                lt_ref, ones_ref, l288_ref, lin_ref,
                out_ref,
                x1s, y1s, x2s, y2s, ars, mbs):
    f32 = jnp.float32
    lane = jax.lax.broadcasted_iota(jnp.int32, (1, _LANES), 1)
    wa = wa_ref[...]
    ha = ha_ref[...]
    cxa = cxa_ref[...]
    cya = cya_ref[...]
    lin = lin_ref[...]

    for b in range(_B):
        dx = dx_ref[b]
        dy = dy_ref[b]
        dw = dw_ref[b]
        dh = dh_ref[b]
        pcx = dx * wa + cxa
        pcy = dy * ha + cya
        pw = jnp.exp(dw) * wa
        ph = jnp.exp(dh) * ha
        x1 = pcx - f32(0.5) * pw
        y1 = pcy - f32(0.5) * ph
        x2 = pcx + f32(0.5) * pw
        y2 = pcy + f32(0.5) * ph
        imh = im_ref[b, 0] - f32(1.0)
        imw = im_ref[b, 1] - f32(1.0)
        x1 = jnp.minimum(jnp.maximum(x1, f32(0.0)), imw)
        y1 = jnp.minimum(jnp.maximum(y1, f32(0.0)), imh)
        x2 = jnp.minimum(jnp.maximum(x2, f32(0.0)), imw)
        y2 = jnp.minimum(jnp.maximum(y2, f32(0.0)), imh)
        x1s[b] = x1
        y1s[b] = y1
        x2s[b] = x2
        y2s[b] = y2
        ars[b] = (x2 - x1) * (y2 - y1)

        sb = sb_ref[b]
        hi0 = jnp.max(sb) + 1

        def bs_body(_, lohi):
            lo, hi = lohi
            mid = jax.lax.shift_right_logical(lo + hi, 1)
            cnt = jnp.sum(jnp.where(sb >= mid, f32(1.0), f32(0.0)))
            ge = cnt >= f32(_PRE)
            return (jnp.where(ge, mid, lo), jnp.where(ge, hi, mid))

        tau, _ = jax.lax.fori_loop(0, 31, bs_body, (jnp.int32(0), hi0))
        cgt = jnp.sum(jnp.where(sb > tau, f32(1.0), f32(0.0)))
        need = f32(_PRE) - cgt
        tie = jnp.where(sb == tau, f32(1.0), f32(0.0))
        lanep = jnp.dot(tie, lt_ref[...], preferred_element_type=f32)
        rowt = jnp.dot(tie, ones_ref[...], preferred_element_type=f32)
        rowp = jnp.dot(l288_ref[...], rowt, preferred_element_type=f32)
        rank = lanep + rowp
        valid = (sb > tau) | ((sb == tau) & (rank < need))
        mbs[b] = jnp.where(valid, sb, jnp.int32(-1))

    def nms_body(j, carry):
        for b in range(_B):
            mb = mbs[b]
            m = jnp.max(mb)
            has = m >= 0
            idx_f = jnp.min(jnp.where(mb == m, lin, f32(3.9e7)))
            idxi = idx_f.astype(jnp.int32)
            r = jax.lax.shift_right_logical(idxi, 7)
            li = jax.lax.bitwise_and(idxi, 127)
            oh = jnp.where(lane == li, f32(1.0), f32(0.0))
            x1c = jnp.sum(x1s[b, pl.ds(r, 1), :] * oh)
            y1c = jnp.sum(y1s[b, pl.ds(r, 1), :] * oh)
            x2c = jnp.sum(x2s[b, pl.ds(r, 1), :] * oh)
            y2c = jnp.sum(y2s[b, pl.ds(r, 1), :] * oh)
            arc = jnp.sum(ars[b, pl.ds(r, 1), :] * oh)
            xx1 = jnp.maximum(x1c, x1s[b])
            yy1 = jnp.maximum(y1c, y1s[b])
            xx2 = jnp.minimum(x2c, x2s[b])
            yy2 = jnp.minimum(y2c, y2s[b])
            iw = jnp.maximum(xx2 - xx1, f32(0.0))
            ih = jnp.maximum(yy2 - yy1, f32(0.0))
            inter = iw * ih
            denom = arc + ars[b] - inter + f32(1e-9)
            iou = inter / denom
            kill = (iou > f32(_THRESH)) | (lin == idx_f)
            mbs[b] = jnp.where(kill & has, jnp.int32(-1), mb)
            valf = jnp.where(has, f32(1.0), f32(0.0))
            row = jnp.where(lane == 0, f32(b), f32(0.0))
            row = row + jnp.where(lane == 1, x1c * valf, f32(0.0))
            row = row + jnp.where(lane == 2, y1c * valf, f32(0.0))
            row = row + jnp.where(lane == 3, x2c * valf, f32(0.0))
            row = row + jnp.where(lane == 4, y2c * valf, f32(0.0))
            out_ref[b, pl.ds(j, 1), :] = row
        return carry

    jax.lax.fori_loop(0, _POST, nms_body, jnp.int32(0))


def kernel(scores, bbox_deltas, im_info):
    sc = jnp.transpose(scores, (0, 2, 3, 1)).reshape(_B, _ROWS, _LANES)
    sb = jax.lax.bitcast_convert_type(sc, jnp.int32)
    d = jnp.transpose(bbox_deltas, (0, 2, 3, 1)).reshape(_B, _H * _W * _A, 4)
    dx = d[..., 0].reshape(_B, _ROWS, _LANES)
    dy = d[..., 1].reshape(_B, _ROWS, _LANES)
    dw = d[..., 2].reshape(_B, _ROWS, _LANES)
    dh = d[..., 3].reshape(_B, _ROWS, _LANES)
    im_pad = jnp.zeros((_B, _LANES), jnp.float32).at[:, :3].set(im_info)

    out = pl.pallas_call(
        _nms_kernel,
        out_shape=jax.ShapeDtypeStruct((_B, 304, _LANES), jnp.float32),
        scratch_shapes=[pltpu.VMEM((_B, _ROWS, _LANES), jnp.float32)] * 5
        + [pltpu.VMEM((_B, _ROWS, _LANES), jnp.int32)],
    )(sb, dx, dy, dw, dh, im_pad,
      _WA, _HA, _CXA, _CYA, _LT128, _ONES128, _L288, _LIN)
    return out[:, :_POST, :5]
